# Initial kernel scaffold; baseline (speedup 1.0000x reference)
import jax
import jax.numpy as jnp
from jax.experimental import pallas as pl


def kernel(x, pos, edge_index, edge_attr, batch, params):
    raise NotImplementedError("write your pallas kernel here")


# speedup vs baseline: 2.0022x; 2.0022x over previous
import functools

import jax
import jax.numpy as jnp
from jax import lax
from jax.experimental import pallas as pl
from jax.experimental.pallas import tpu as pltpu
from jax.experimental.pallas import tpu_sc as plsc

N = 10000
E = 320000
H = 128
G = 64
CP = 16
TW = H + CP
EPS = 1e-8
N_LAYERS = 4

NC = 2
NS = 16
NW = NC * NS
CHUNK = 80
EW = E // NW
NCHUNK = EW // CHUNK

BN = 1000
BE = 2000

_F32 = jnp.float32


def _silu(v):
    return v * jax.nn.sigmoid(v)


def _full_spec(shape):
    return pl.BlockSpec(shape, lambda *_: tuple(0 for _ in shape))


def _make_sc_gather():
    mesh = plsc.VectorSubcoreMesh(
        core_axis_name="c", subcore_axis_name="s",
        num_cores=NC, num_subcores=NS)

    @functools.partial(
        pl.kernel,
        out_type=(jax.ShapeDtypeStruct((E, TW), _F32),
                  jax.ShapeDtypeStruct((E, TW), _F32)),
        mesh=mesh,
        scratch_types=(pltpu.VMEM((CHUNK,), jnp.int32),
                       pltpu.VMEM((CHUNK,), jnp.int32),
                       pltpu.VMEM((CHUNK, TW), _F32),
                       pltpu.VMEM((CHUNK, TW), _F32),
                       pltpu.SemaphoreType.DMA,
                       pltpu.SemaphoreType.DMA),
        compiler_params=pltpu.CompilerParams(use_tc_tiling_on_sc=False),
    )
    def gather(ta, tb, row_h, col_h, g1, g2, idx1, idx2, buf1, buf2, s1, s2):
        wid = lax.axis_index("s") * NC + lax.axis_index("c")
        base0 = wid * EW

        def body(j, carry):
            base = base0 + j * CHUNK
            pltpu.sync_copy(row_h.at[pl.ds(base, CHUNK)], idx1)
            pltpu.sync_copy(col_h.at[pl.ds(base, CHUNK)], idx2)
            c1 = pltpu.async_copy(ta.at[idx1], buf1, s1)
            c2 = pltpu.async_copy(tb.at[idx2], buf2, s2)
            c1.wait()
            c2.wait()
            pltpu.sync_copy(buf1, g1.at[pl.ds(base, CHUNK)])
            pltpu.sync_copy(buf2, g2.at[pl.ds(base, CHUNK)])
            return carry

        lax.fori_loop(0, NCHUNK, body, 0)

    return gather


def _make_sc_scatter(n_items, n_rows, chunk, workers):
    ew = n_items // workers
    nchunk = ew // chunk
    rps = n_rows // NS
    mesh = plsc.VectorSubcoreMesh(
        core_axis_name="c", subcore_axis_name="s",
        num_cores=NC, num_subcores=NS)

    @functools.partial(
        pl.kernel,
        out_type=jax.ShapeDtypeStruct((NC, n_rows, TW), _F32),
        mesh=mesh,
        scratch_types=(pltpu.VMEM((chunk,), jnp.int32),
                       pltpu.VMEM((chunk, TW), _F32),
                       pltpu.VMEM_SHARED((n_rows, TW), _F32)),
        compiler_params=pltpu.CompilerParams(use_tc_tiling_on_sc=False),
    )
    def scatter(pay_h, idx_h, zero_h, out_h, idx_v, buf, acc):
        cid = lax.axis_index("c")
        sid = lax.axis_index("s")
        wid = sid * NC + cid
        pltpu.sync_copy(zero_h.at[pl.ds(sid * rps, rps)],
                        acc.at[pl.ds(sid * rps, rps)])
        plsc.subcore_barrier()

        @pl.when(wid < workers)
        def _():
            def body(j, carry):
                base = wid * ew + j * chunk
                pltpu.sync_copy(idx_h.at[pl.ds(base, chunk)], idx_v)
                pltpu.sync_copy(pay_h.at[pl.ds(base, chunk)], buf)
                pltpu.sync_copy(buf, acc.at[idx_v], add=True)
                return carry

            lax.fori_loop(0, nchunk, body, 0)

        plsc.subcore_barrier()
        pltpu.sync_copy(acc.at[pl.ds(sid * rps, rps)],
                        out_h.at[cid, pl.ds(sid * rps, rps)])

    return scatter


def _embed_body(x_ref, cp_ref, wemb, bemb, wr, br, wc,
                h_ref, a_ref, b_ref):
    hv = jnp.dot(x_ref[...], wemb[...], preferred_element_type=_F32) + bemb[...]
    cpv = cp_ref[...]
    h_ref[...] = hv
    a_ref[...] = jnp.concatenate(
        [jnp.dot(hv, wr[...], preferred_element_type=_F32) + br[...], cpv], axis=1)
    b_ref[...] = jnp.concatenate(
        [jnp.dot(hv, wc[...], preferred_element_type=_F32), -cpv], axis=1)


def _tc_embed(x, cp, wemb, bemb, wr, br, wc):
    return pl.pallas_call(
        _embed_body,
        grid=(N // BN,),
        in_specs=[pl.BlockSpec((BN, H), lambda i: (i, 0)),
                  pl.BlockSpec((BN, CP), lambda i: (i, 0)),
                  _full_spec((H, H)), _full_spec((1, H)),
                  _full_spec((H, H)), _full_spec((1, H)),
                  _full_spec((H, H))],
        out_specs=[pl.BlockSpec((BN, H), lambda i: (i, 0)),
                   pl.BlockSpec((BN, TW), lambda i: (i, 0)),
                   pl.BlockSpec((BN, TW), lambda i: (i, 0))],
        out_shape=[jax.ShapeDtypeStruct((N, H), _F32),
                   jax.ShapeDtypeStruct((N, TW), _F32),
                   jax.ShapeDtypeStruct((N, TW), _F32)],
    )(x, cp, wemb, bemb, wr, br, wc)


def _edge_body(g1_ref, g2_ref, ea_ref, wrad, wea, w2, b2, watt, batt,
               wc1, bc1, wc2, out_ref):
    g = g1_ref[...] + g2_ref[...]
    hsum = g[:, :H]
    cd = g[:, H:]
    radial = jnp.sum(cd * cd, axis=1, keepdims=True)
    t = hsum + radial * wrad[...] + jnp.dot(
        ea_ref[...], wea[...], preferred_element_type=_F32)
    m = _silu(t)
    m = _silu(jnp.dot(m, w2[...], preferred_element_type=_F32) + b2[...])
    att = jax.nn.sigmoid(
        jnp.sum(m * watt[...], axis=1, keepdims=True) + batt[0, 0])
    m = m * att
    cmid = _silu(jnp.dot(m, wc1[...], preferred_element_type=_F32) + bc1[...])
    cval = jnp.sum(cmid * wc2[...], axis=1, keepdims=True)
    scale = cval / (jnp.sqrt(radial) + EPS)
    ones_col = (lax.broadcasted_iota(jnp.int32, (BE, CP), 1) == 3).astype(_F32)
    out_ref[...] = jnp.concatenate([m, cd * scale + ones_col], axis=1)


def _tc_edge(g1, g2, ea, wrad, wea, w2, b2, watt, batt, wc1, bc1, wc2):
    return pl.pallas_call(
        _edge_body,
        grid=(E // BE,),
        in_specs=[pl.BlockSpec((BE, TW), lambda i: (i, 0)),
                  pl.BlockSpec((BE, TW), lambda i: (i, 0)),
                  pl.BlockSpec((BE, 4), lambda i: (i, 0)),
                  _full_spec((1, H)), _full_spec((4, H)),
                  _full_spec((H, H)), _full_spec((1, H)),
                  _full_spec((1, H)), _full_spec((1, 1)),
                  _full_spec((H, H)), _full_spec((1, H)),
                  _full_spec((1, H))],
        out_specs=pl.BlockSpec((BE, TW), lambda i: (i, 0)),
        out_shape=jax.ShapeDtypeStruct((E, TW), _F32),
    )(g1, g2, ea, wrad, wea, w2, b2, watt, batt, wc1, bc1, wc2)


def _node_common(acc_ref, h_ref, w1a, w1b, b1, w2, b2):
    a = acc_ref[0] + acc_ref[1]
    nagg = a[:, :H]
    ctail = a[:, H:]
    lane = lax.broadcasted_iota(jnp.int32, (BN, CP), 1)
    cnt = jnp.sum(jnp.where(lane == 3, ctail, 0.0), axis=1, keepdims=True)
    upd = jnp.where(lane < 3, ctail, 0.0) / jnp.maximum(cnt, 1.0)
    hv = h_ref[...]
    t = _silu(jnp.dot(hv, w1a[...], preferred_element_type=_F32)
              + jnp.dot(nagg, w1b[...], preferred_element_type=_F32)
              + b1[...])
    hnew = hv + jnp.dot(t, w2[...], preferred_element_type=_F32) + b2[...]
    return hnew, upd


def _node_body(acc_ref, h_ref, cp_ref, w1a, w1b, b1, w2, b2, wrn, brn, wcn,
               ho_ref, co_ref, a_ref, bo_ref):
    hnew, upd = _node_common(acc_ref, h_ref, w1a, w1b, b1, w2, b2)
    cnew = cp_ref[...] + upd
    ho_ref[...] = hnew
    co_ref[...] = cnew
    a_ref[...] = jnp.concatenate(
        [jnp.dot(hnew, wrn[...], preferred_element_type=_F32) + brn[...], cnew],
        axis=1)
    bo_ref[...] = jnp.concatenate(
        [jnp.dot(hnew, wcn[...], preferred_element_type=_F32), -cnew], axis=1)


def _tc_node(acc, h, cp, w1a, w1b, b1, w2, b2, wrn, brn, wcn):
    return pl.pallas_call(
        _node_body,
        grid=(N // BN,),
        in_specs=[pl.BlockSpec((NC, BN, TW), lambda i: (0, i, 0)),
                  pl.BlockSpec((BN, H), lambda i: (i, 0)),
                  pl.BlockSpec((BN, CP), lambda i: (i, 0)),
                  _full_spec((H, H)), _full_spec((H, H)), _full_spec((1, H)),
                  _full_spec((H, H)), _full_spec((1, H)),
                  _full_spec((H, H)), _full_spec((1, H)), _full_spec((H, H))],
        out_specs=[pl.BlockSpec((BN, H), lambda i: (i, 0)),
                   pl.BlockSpec((BN, CP), lambda i: (i, 0)),
                   pl.BlockSpec((BN, TW), lambda i: (i, 0)),
                   pl.BlockSpec((BN, TW), lambda i: (i, 0))],
        out_shape=[jax.ShapeDtypeStruct((N, H), _F32),
                   jax.ShapeDtypeStruct((N, CP), _F32),
                   jax.ShapeDtypeStruct((N, TW), _F32),
                   jax.ShapeDtypeStruct((N, TW), _F32)],
    )(acc, h, cp, w1a, w1b, b1, w2, b2, wrn, brn, wcn)


def _final_body(acc_ref, h_ref, w1a, w1b, b1, w2, b2, weo, beo, nf_ref):
    hnew, _ = _node_common(acc_ref, h_ref, w1a, w1b, b1, w2, b2)
    nf = jnp.dot(hnew, weo[...], preferred_element_type=_F32) + beo[...]
    ones_col = (lax.broadcasted_iota(jnp.int32, (BN, CP), 1) == 3).astype(_F32)
    nf_ref[...] = jnp.concatenate([nf, ones_col], axis=1)


def _tc_final(acc, h, w1a, w1b, b1, w2, b2, weo, beo):
    return pl.pallas_call(
        _final_body,
        grid=(N // BN,),
        in_specs=[pl.BlockSpec((NC, BN, TW), lambda i: (0, i, 0)),
                  pl.BlockSpec((BN, H), lambda i: (i, 0)),
                  _full_spec((H, H)), _full_spec((H, H)), _full_spec((1, H)),
                  _full_spec((H, H)), _full_spec((1, H)),
                  _full_spec((H, H)), _full_spec((1, H))],
        out_specs=pl.BlockSpec((BN, TW), lambda i: (i, 0)),
        out_shape=jax.ShapeDtypeStruct((N, TW), _F32),
    )(acc, h, w1a, w1b, b1, w2, b2, weo, beo)


def _pool_body(acc_ref, w1, b1, w2, b2, out_ref):
    a = acc_ref[0] + acc_ref[1]
    gsum = a[:, :H]
    tail = a[:, H:]
    lane = lax.broadcasted_iota(jnp.int32, (G, CP), 1)
    cnt = jnp.sum(jnp.where(lane == 3, tail, 0.0), axis=1, keepdims=True)
    pooled = gsum / jnp.maximum(cnt, 1.0)
    o = _silu(jnp.dot(pooled, w1[...], preferred_element_type=_F32) + b1[...])
    out_ref[...] = jnp.sum(o * w2[...], axis=1, keepdims=True) + b2[0, 0]


def _tc_pool(acc, w1, b1, w2, b2):
    return pl.pallas_call(
        _pool_body,
        grid=(1,),
        in_specs=[_full_spec((NC, G, TW)),
                  _full_spec((H, H)), _full_spec((1, H)),
                  _full_spec((1, H)), _full_spec((1, 1))],
        out_specs=_full_spec((G, 1)),
        out_shape=jax.ShapeDtypeStruct((G, 1), _F32),
    )(acc, w1, b1, w2, b2)


def kernel(x, pos, edge_index, edge_attr, batch, params):
    row = edge_index[0]
    col = edge_index[1]
    cp0 = jnp.pad(pos, ((0, 0), (0, CP - 3)))
    zeros_n = jnp.zeros((N, TW), _F32)
    zeros_g = jnp.zeros((G, TW), _F32)

    layers = params['layers']

    def e1_split(lp):
        w = lp['e1']['W']
        return (w[:H], w[H:2 * H], w[2 * H:2 * H + 1], w[2 * H + 1:],
                lp['e1']['b'].reshape(1, H))

    sc_gather = _make_sc_gather()
    edge_scatter = _make_sc_scatter(E, N, CHUNK, NW)
    pool_scatter = _make_sc_scatter(N, G, CHUNK, 25)

    wr0, wc0, _, _, br0 = e1_split(layers[0])
    h, ta, tb = _tc_embed(
        x, cp0, params['emb_in']['W'], params['emb_in']['b'].reshape(1, H),
        wr0, br0, wc0)
    coord = cp0

    nf = None
    for l in range(N_LAYERS):
        lp = layers[l]
        _, _, wrad, wea, _ = e1_split(lp)
        g1, g2 = sc_gather(ta, tb, row, col)
        pay = _tc_edge(
            g1, g2, edge_attr, wrad, wea,
            lp['e2']['W'], lp['e2']['b'].reshape(1, H),
            lp['att']['W'].reshape(1, H), lp['att']['b'].reshape(1, 1),
            lp['c1']['W'], lp['c1']['b'].reshape(1, H),
            lp['c2']['W'].reshape(1, H))
        acc = edge_scatter(pay, row, zeros_n)
        n1w = lp['n1']['W']
        nodew = (n1w[:H], n1w[H:], lp['n1']['b'].reshape(1, H),
                 lp['n2']['W'], lp['n2']['b'].reshape(1, H))
        if l < N_LAYERS - 1:
            wrn, wcn, _, _, brn = e1_split(layers[l + 1])
            h, coord, ta, tb = _tc_node(acc, h, coord, *nodew, wrn, brn, wcn)
        else:
            nf = _tc_final(acc, h, *nodew,
                           params['emb_out']['W'],
                           params['emb_out']['b'].reshape(1, H))

    pacc = pool_scatter(nf, batch, zeros_g)
    return _tc_pool(
        pacc, params['out1']['W'], params['out1']['b'].reshape(1, H),
        params['out2']['W'].reshape(1, H), params['out2']['b'].reshape(1, 1))


# speedup vs baseline: 2.2459x; 1.1217x over previous
import functools

import jax
import jax.numpy as jnp
from jax import lax
from jax.experimental import pallas as pl
from jax.experimental.pallas import tpu as pltpu
from jax.experimental.pallas import tpu_sc as plsc

N = 10000
E = 320000
H = 128
G = 64
CP = 16
TW = H + CP
EPS = 1e-8
N_LAYERS = 4

NC = 2
NS = 16
NW = NC * NS
CHUNK = 80
SCHUNK = 40
GK = 5
EW = E // NW
NCHUNK = EW // CHUNK

BN = 1000
BE = 2000

_F32 = jnp.float32


def _silu(v):
    return v * jax.nn.sigmoid(v)


def _full_spec(shape):
    return pl.BlockSpec(shape, lambda *_: tuple(0 for _ in shape))


def _make_sc_gather():
    mesh = plsc.VectorSubcoreMesh(
        core_axis_name="c", subcore_axis_name="s",
        num_cores=NC, num_subcores=NS)

    @functools.partial(
        pl.kernel,
        out_type=(jax.ShapeDtypeStruct((E, TW), _F32),
                  jax.ShapeDtypeStruct((E, TW), _F32)),
        mesh=mesh,
        scratch_types=(pltpu.VMEM((GK, CHUNK), jnp.int32),
                       pltpu.VMEM((GK, CHUNK), jnp.int32),
                       pltpu.VMEM((GK * CHUNK, TW), _F32),
                       pltpu.VMEM((GK * CHUNK, TW), _F32))
                      + (pltpu.SemaphoreType.DMA,) * (2 * GK),
        compiler_params=pltpu.CompilerParams(use_tc_tiling_on_sc=False),
    )
    def gather(ta, tb, row2, col2, g1, g2, idxr, idxc, bufa, bufb, *sems):
        gs, ws = sems[:GK], sems[GK:]
        wid = lax.axis_index("s") * NC + lax.axis_index("c")
        base0 = wid * EW
        crow0 = wid * NCHUNK

        def body(g, carry):
            crow = crow0 + g * GK
            pltpu.sync_copy(row2.at[pl.ds(crow, GK)], idxr)
            pltpu.sync_copy(col2.at[pl.ds(crow, GK)], idxc)
            cps = []
            for b in range(GK):
                sl = pl.ds(b * CHUNK, CHUNK)
                cps.append(pltpu.async_copy(ta.at[idxr.at[b]], bufa.at[sl], gs[b]))
                cps.append(pltpu.async_copy(tb.at[idxc.at[b]], bufb.at[sl], gs[b]))
            wps = []
            for b in range(GK):
                cps[2 * b].wait()
                cps[2 * b + 1].wait()
                sl = pl.ds(b * CHUNK, CHUNK)
                ebase = base0 + (g * GK + b) * CHUNK
                wps.append(pltpu.async_copy(bufa.at[sl], g1.at[pl.ds(ebase, CHUNK)], ws[b]))
                wps.append(pltpu.async_copy(bufb.at[sl], g2.at[pl.ds(ebase, CHUNK)], ws[b]))
            for w in wps:
                w.wait()
            return carry

        lax.fori_loop(0, NCHUNK // GK, body, 0)

    return gather


def _make_sc_scatter(n_items, n_rows, chunk, workers):
    ew = n_items // workers
    nchunk = ew // chunk
    gk = min(GK, nchunk)
    ngroup = nchunk // gk
    rps = n_rows // NS
    mesh = plsc.VectorSubcoreMesh(
        core_axis_name="c", subcore_axis_name="s",
        num_cores=NC, num_subcores=NS)

    @functools.partial(
        pl.kernel,
        out_type=jax.ShapeDtypeStruct((NC, n_rows, TW), _F32),
        mesh=mesh,
        scratch_types=(pltpu.VMEM((gk, chunk), jnp.int32),
                       pltpu.VMEM((gk * chunk, TW), _F32))
                      + (pltpu.SemaphoreType.DMA,) * (2 * gk)
                      + (pltpu.VMEM_SHARED((n_rows, TW), _F32),),
        compiler_params=pltpu.CompilerParams(use_tc_tiling_on_sc=False),
    )
    def scatter(pay_h, idx2_h, zero_h, out_h, idx_v, buf, *rest):
        ls, ss, acc = rest[:gk], rest[gk:2 * gk], rest[2 * gk]
        cid = lax.axis_index("c")
        sid = lax.axis_index("s")
        wid = sid * NC + cid
        pltpu.sync_copy(zero_h.at[pl.ds(sid * rps, rps)],
                        acc.at[pl.ds(sid * rps, rps)])
        plsc.subcore_barrier()

        @pl.when(wid < workers)
        def _():
            def body(g, carry):
                crow = wid * nchunk + g * gk
                pltpu.sync_copy(idx2_h.at[pl.ds(crow, gk)], idx_v)
                lps = []
                for b in range(gk):
                    ebase = wid * ew + (g * gk + b) * chunk
                    lps.append(pltpu.async_copy(
                        pay_h.at[pl.ds(ebase, chunk)],
                        buf.at[pl.ds(b * chunk, chunk)], ls[b]))
                sps = []
                for b in range(gk):
                    lps[b].wait()
                    sps.append(pltpu.async_copy(
                        buf.at[pl.ds(b * chunk, chunk)],
                        acc.at[idx_v.at[b]], ss[b], add=True))
                for s_ in sps:
                    s_.wait()
                return carry

            lax.fori_loop(0, ngroup, body, 0)

        plsc.subcore_barrier()
        pltpu.sync_copy(acc.at[pl.ds(sid * rps, rps)],
                        out_h.at[cid, pl.ds(sid * rps, rps)])

    return scatter


def _embed_body(x_ref, cp_ref, wemb, bemb, wr, br, wc,
                h_ref, a_ref, b_ref):
    hv = jnp.dot(x_ref[...], wemb[...], preferred_element_type=_F32) + bemb[...]
    cpv = cp_ref[...]
    h_ref[...] = hv
    a_ref[...] = jnp.concatenate(
        [jnp.dot(hv, wr[...], preferred_element_type=_F32) + br[...], cpv], axis=1)
    b_ref[...] = jnp.concatenate(
        [jnp.dot(hv, wc[...], preferred_element_type=_F32), -cpv], axis=1)


def _tc_embed(x, cp, wemb, bemb, wr, br, wc):
    return pl.pallas_call(
        _embed_body,
        grid=(N // BN,),
        in_specs=[pl.BlockSpec((BN, H), lambda i: (i, 0)),
                  pl.BlockSpec((BN, CP), lambda i: (i, 0)),
                  _full_spec((H, H)), _full_spec((1, H)),
                  _full_spec((H, H)), _full_spec((1, H)),
                  _full_spec((H, H))],
        out_specs=[pl.BlockSpec((BN, H), lambda i: (i, 0)),
                   pl.BlockSpec((BN, TW), lambda i: (i, 0)),
                   pl.BlockSpec((BN, TW), lambda i: (i, 0))],
        out_shape=[jax.ShapeDtypeStruct((N, H), _F32),
                   jax.ShapeDtypeStruct((N, TW), _F32),
                   jax.ShapeDtypeStruct((N, TW), _F32)],
    )(x, cp, wemb, bemb, wr, br, wc)


def _edge_body(g1_ref, g2_ref, ea_ref, wrad, wea, w2, b2, watt, batt,
               wc1, bc1, wc2, out_ref):
    g = g1_ref[...] + g2_ref[...]
    hsum = g[:, :H]
    cd = g[:, H:]
    radial = jnp.sum(cd * cd, axis=1, keepdims=True)
    t = hsum + radial * wrad[...] + jnp.dot(
        ea_ref[...], wea[...], preferred_element_type=_F32)
    m = _silu(t)
    m = _silu(jnp.dot(m, w2[...], preferred_element_type=_F32) + b2[...])
    att = jax.nn.sigmoid(
        jnp.sum(m * watt[...], axis=1, keepdims=True) + batt[0, 0])
    m = m * att
    cmid = _silu(jnp.dot(m, wc1[...], preferred_element_type=_F32) + bc1[...])
    cval = jnp.sum(cmid * wc2[...], axis=1, keepdims=True)
    scale = cval / (jnp.sqrt(radial) + EPS)
    ones_col = (lax.broadcasted_iota(jnp.int32, (BE, CP), 1) == 3).astype(_F32)
    out_ref[...] = jnp.concatenate([m, cd * scale + ones_col], axis=1)


def _tc_edge(g1, g2, ea, wrad, wea, w2, b2, watt, batt, wc1, bc1, wc2):
    return pl.pallas_call(
        _edge_body,
        grid=(E // BE,),
        in_specs=[pl.BlockSpec((BE, TW), lambda i: (i, 0)),
                  pl.BlockSpec((BE, TW), lambda i: (i, 0)),
                  pl.BlockSpec((BE, 4), lambda i: (i, 0)),
                  _full_spec((1, H)), _full_spec((4, H)),
                  _full_spec((H, H)), _full_spec((1, H)),
                  _full_spec((1, H)), _full_spec((1, 1)),
                  _full_spec((H, H)), _full_spec((1, H)),
                  _full_spec((1, H))],
        out_specs=pl.BlockSpec((BE, TW), lambda i: (i, 0)),
        out_shape=jax.ShapeDtypeStruct((E, TW), _F32),
    )(g1, g2, ea, wrad, wea, w2, b2, watt, batt, wc1, bc1, wc2)


def _node_common(acc_ref, h_ref, w1a, w1b, b1, w2, b2):
    a = acc_ref[0] + acc_ref[1]
    nagg = a[:, :H]
    ctail = a[:, H:]
    lane = lax.broadcasted_iota(jnp.int32, (BN, CP), 1)
    cnt = jnp.sum(jnp.where(lane == 3, ctail, 0.0), axis=1, keepdims=True)
    upd = jnp.where(lane < 3, ctail, 0.0) / jnp.maximum(cnt, 1.0)
    hv = h_ref[...]
    t = _silu(jnp.dot(hv, w1a[...], preferred_element_type=_F32)
              + jnp.dot(nagg, w1b[...], preferred_element_type=_F32)
              + b1[...])
    hnew = hv + jnp.dot(t, w2[...], preferred_element_type=_F32) + b2[...]
    return hnew, upd


def _node_body(acc_ref, h_ref, cp_ref, w1a, w1b, b1, w2, b2, wrn, brn, wcn,
               ho_ref, co_ref, a_ref, bo_ref):
    hnew, upd = _node_common(acc_ref, h_ref, w1a, w1b, b1, w2, b2)
    cnew = cp_ref[...] + upd
    ho_ref[...] = hnew
    co_ref[...] = cnew
    a_ref[...] = jnp.concatenate(
        [jnp.dot(hnew, wrn[...], preferred_element_type=_F32) + brn[...], cnew],
        axis=1)
    bo_ref[...] = jnp.concatenate(
        [jnp.dot(hnew, wcn[...], preferred_element_type=_F32), -cnew], axis=1)


def _tc_node(acc, h, cp, w1a, w1b, b1, w2, b2, wrn, brn, wcn):
    return pl.pallas_call(
        _node_body,
        grid=(N // BN,),
        in_specs=[pl.BlockSpec((NC, BN, TW), lambda i: (0, i, 0)),
                  pl.BlockSpec((BN, H), lambda i: (i, 0)),
                  pl.BlockSpec((BN, CP), lambda i: (i, 0)),
                  _full_spec((H, H)), _full_spec((H, H)), _full_spec((1, H)),
                  _full_spec((H, H)), _full_spec((1, H)),
                  _full_spec((H, H)), _full_spec((1, H)), _full_spec((H, H))],
        out_specs=[pl.BlockSpec((BN, H), lambda i: (i, 0)),
                   pl.BlockSpec((BN, CP), lambda i: (i, 0)),
                   pl.BlockSpec((BN, TW), lambda i: (i, 0)),
                   pl.BlockSpec((BN, TW), lambda i: (i, 0))],
        out_shape=[jax.ShapeDtypeStruct((N, H), _F32),
                   jax.ShapeDtypeStruct((N, CP), _F32),
                   jax.ShapeDtypeStruct((N, TW), _F32),
                   jax.ShapeDtypeStruct((N, TW), _F32)],
    )(acc, h, cp, w1a, w1b, b1, w2, b2, wrn, brn, wcn)


def _final_body(acc_ref, h_ref, w1a, w1b, b1, w2, b2, weo, beo, nf_ref):
    hnew, _ = _node_common(acc_ref, h_ref, w1a, w1b, b1, w2, b2)
    nf = jnp.dot(hnew, weo[...], preferred_element_type=_F32) + beo[...]
    ones_col = (lax.broadcasted_iota(jnp.int32, (BN, CP), 1) == 3).astype(_F32)
    nf_ref[...] = jnp.concatenate([nf, ones_col], axis=1)


def _tc_final(acc, h, w1a, w1b, b1, w2, b2, weo, beo):
    return pl.pallas_call(
        _final_body,
        grid=(N // BN,),
        in_specs=[pl.BlockSpec((NC, BN, TW), lambda i: (0, i, 0)),
                  pl.BlockSpec((BN, H), lambda i: (i, 0)),
                  _full_spec((H, H)), _full_spec((H, H)), _full_spec((1, H)),
                  _full_spec((H, H)), _full_spec((1, H)),
                  _full_spec((H, H)), _full_spec((1, H))],
        out_specs=pl.BlockSpec((BN, TW), lambda i: (i, 0)),
        out_shape=jax.ShapeDtypeStruct((N, TW), _F32),
    )(acc, h, w1a, w1b, b1, w2, b2, weo, beo)


def _pool_body(acc_ref, w1, b1, w2, b2, out_ref):
    a = acc_ref[0] + acc_ref[1]
    gsum = a[:, :H]
    tail = a[:, H:]
    lane = lax.broadcasted_iota(jnp.int32, (G, CP), 1)
    cnt = jnp.sum(jnp.where(lane == 3, tail, 0.0), axis=1, keepdims=True)
    pooled = gsum / jnp.maximum(cnt, 1.0)
    o = _silu(jnp.dot(pooled, w1[...], preferred_element_type=_F32) + b1[...])
    out_ref[...] = jnp.sum(o * w2[...], axis=1, keepdims=True) + b2[0, 0]


def _tc_pool(acc, w1, b1, w2, b2):
    return pl.pallas_call(
        _pool_body,
        grid=(1,),
        in_specs=[_full_spec((NC, G, TW)),
                  _full_spec((H, H)), _full_spec((1, H)),
                  _full_spec((1, H)), _full_spec((1, 1))],
        out_specs=_full_spec((G, 1)),
        out_shape=jax.ShapeDtypeStruct((G, 1), _F32),
    )(acc, w1, b1, w2, b2)


def kernel(x, pos, edge_index, edge_attr, batch, params):
    row = edge_index[0]
    row2 = row.reshape(E // CHUNK, CHUNK)
    col2 = edge_index[1].reshape(E // CHUNK, CHUNK)
    rows2 = row.reshape(E // SCHUNK, SCHUNK)
    batch2 = batch.reshape(N // SCHUNK, SCHUNK)
    cp0 = jnp.pad(pos, ((0, 0), (0, CP - 3)))
    zeros_n = jnp.zeros((N, TW), _F32)
    zeros_g = jnp.zeros((G, TW), _F32)

    layers = params['layers']

    def e1_split(lp):
        w = lp['e1']['W']
        return (w[:H], w[H:2 * H], w[2 * H:2 * H + 1], w[2 * H + 1:],
                lp['e1']['b'].reshape(1, H))

    sc_gather = _make_sc_gather()
    edge_scatter = _make_sc_scatter(E, N, SCHUNK, NW)
    pool_scatter = _make_sc_scatter(N, G, SCHUNK, 25)

    wr0, wc0, _, _, br0 = e1_split(layers[0])
    h, ta, tb = _tc_embed(
        x, cp0, params['emb_in']['W'], params['emb_in']['b'].reshape(1, H),
        wr0, br0, wc0)
    coord = cp0

    nf = None
    for l in range(N_LAYERS):
        lp = layers[l]
        _, _, wrad, wea, _ = e1_split(lp)
        g1, g2 = sc_gather(ta, tb, row2, col2)
        pay = _tc_edge(
            g1, g2, edge_attr, wrad, wea,
            lp['e2']['W'], lp['e2']['b'].reshape(1, H),
            lp['att']['W'].reshape(1, H), lp['att']['b'].reshape(1, 1),
            lp['c1']['W'], lp['c1']['b'].reshape(1, H),
            lp['c2']['W'].reshape(1, H))
        acc = edge_scatter(pay, rows2, zeros_n)
        n1w = lp['n1']['W']
        nodew = (n1w[:H], n1w[H:], lp['n1']['b'].reshape(1, H),
                 lp['n2']['W'], lp['n2']['b'].reshape(1, H))
        if l < N_LAYERS - 1:
            wrn, wcn, _, _, brn = e1_split(layers[l + 1])
            h, coord, ta, tb = _tc_node(acc, h, coord, *nodew, wrn, brn, wcn)
        else:
            nf = _tc_final(acc, h, *nodew,
                           params['emb_out']['W'],
                           params['emb_out']['b'].reshape(1, H))

    pacc = pool_scatter(nf, batch2, zeros_g)
    return _tc_pool(
        pacc, params['out1']['W'], params['out1']['b'].reshape(1, H),
        params['out2']['W'].reshape(1, H), params['out2']['b'].reshape(1, 1))


# speedup vs baseline: 3.9019x; 1.7373x over previous
import functools

import jax
import jax.numpy as jnp
from jax import lax
from jax.experimental import pallas as pl
from jax.experimental.pallas import tpu as pltpu
from jax.experimental.pallas import tpu_sc as plsc

N = 10000
E = 320000
H = 128
G = 64
CP = 16
EPS = 1e-8
N_LAYERS = 4

NC = 2
NS = 16
NW = NC * NS
CHUNK = 80
SCHUNK = 40
GK = 5
EW = E // NW
NCHUNK = EW // CHUNK

BN = 1000
BE = 2000

_F32 = jnp.float32


def _silu(v):
    return v * jax.nn.sigmoid(v)


def _full_spec(shape):
    return pl.BlockSpec(shape, lambda *_: tuple(0 for _ in shape))


def _sc_mesh():
    return plsc.VectorSubcoreMesh(
        core_axis_name="c", subcore_axis_name="s",
        num_cores=NC, num_subcores=NS)


def _make_sc_gather():
    @functools.partial(
        pl.kernel,
        out_type=(jax.ShapeDtypeStruct((E, H), _F32),
                  jax.ShapeDtypeStruct((E, H), _F32),
                  jax.ShapeDtypeStruct((E, CP), _F32)),
        mesh=_sc_mesh(),
        scratch_types=(pltpu.VMEM((GK, CHUNK), jnp.int32),
                       pltpu.VMEM((GK, CHUNK), jnp.int32),
                       pltpu.VMEM((GK * CHUNK, H), _F32),
                       pltpu.VMEM((GK * CHUNK, H), _F32),
                       pltpu.VMEM((GK * CHUNK, CP), _F32),
                       pltpu.VMEM((GK * CHUNK, CP), _F32),
                       pltpu.VMEM((GK * CHUNK, CP), _F32))
                      + (pltpu.SemaphoreType.DMA,) * (2 * GK),
        compiler_params=pltpu.CompilerParams(use_tc_tiling_on_sc=False),
    )
    def gather(ta, tb, cta, ctb, row2, col2, g1, g2, cd,
               idxr, idxc, bufa, bufb, bufca, bufcb, bufcd, *sems):
        gs, ws = sems[:GK], sems[GK:]
        wid = lax.axis_index("s") * NC + lax.axis_index("c")
        base0 = wid * EW
        crow0 = wid * NCHUNK

        def body(g, carry):
            crow = crow0 + g * GK
            pltpu.sync_copy(row2.at[pl.ds(crow, GK)], idxr)
            pltpu.sync_copy(col2.at[pl.ds(crow, GK)], idxc)
            cps = []
            for b in range(GK):
                sl = pl.ds(b * CHUNK, CHUNK)
                cps.append((
                    pltpu.async_copy(ta.at[idxr.at[b]], bufa.at[sl], gs[b]),
                    pltpu.async_copy(tb.at[idxc.at[b]], bufb.at[sl], gs[b]),
                    pltpu.async_copy(cta.at[idxr.at[b]], bufca.at[sl], gs[b]),
                    pltpu.async_copy(ctb.at[idxc.at[b]], bufcb.at[sl], gs[b]),
                ))
            wps = []
            for b in range(GK):
                for c in cps[b]:
                    c.wait()
                for i in range(b * CHUNK, (b + 1) * CHUNK):
                    bufcd[i, :] = bufca[i, :] + bufcb[i, :]
                sl = pl.ds(b * CHUNK, CHUNK)
                ebase = base0 + (g * GK + b) * CHUNK
                esl = pl.ds(ebase, CHUNK)
                wps.append(pltpu.async_copy(bufa.at[sl], g1.at[esl], ws[b]))
                wps.append(pltpu.async_copy(bufb.at[sl], g2.at[esl], ws[b]))
                wps.append(pltpu.async_copy(bufcd.at[sl], cd.at[esl], ws[b]))
            for w in wps:
                w.wait()
            return carry

        lax.fori_loop(0, NCHUNK // GK, body, 0)

    return gather


def _make_sc_scatter(n_items, n_rows, chunk, workers):
    ew = n_items // workers
    nchunk = ew // chunk
    gk = min(GK, nchunk)
    ngroup = nchunk // gk
    rps = n_rows // NS

    @functools.partial(
        pl.kernel,
        out_type=(jax.ShapeDtypeStruct((NC, n_rows, H), _F32),
                  jax.ShapeDtypeStruct((NC, n_rows, CP), _F32)),
        mesh=_sc_mesh(),
        scratch_types=(pltpu.VMEM((gk, chunk), jnp.int32),
                       pltpu.VMEM((gk * chunk, H), _F32),
                       pltpu.VMEM((gk * chunk, CP), _F32))
                      + (pltpu.SemaphoreType.DMA,) * (2 * gk)
                      + (pltpu.VMEM_SHARED((n_rows, H), _F32),
                         pltpu.VMEM_SHARED((n_rows, CP), _F32)),
        compiler_params=pltpu.CompilerParams(use_tc_tiling_on_sc=False),
    )
    def scatter(pay_h, pay2_h, idx2_h, zero_h, zero2_h, out_h, out2_h,
                idx_v, buf, buf2, *rest):
        ls, ss = rest[:gk], rest[gk:2 * gk]
        acc, acc2 = rest[2 * gk], rest[2 * gk + 1]
        cid = lax.axis_index("c")
        sid = lax.axis_index("s")
        wid = sid * NC + cid
        rsl = pl.ds(sid * rps, rps)
        pltpu.sync_copy(zero_h.at[rsl], acc.at[rsl])
        pltpu.sync_copy(zero2_h.at[rsl], acc2.at[rsl])
        plsc.subcore_barrier()

        @pl.when(wid < workers)
        def _():
            def body(g, carry):
                crow = wid * nchunk + g * gk
                pltpu.sync_copy(idx2_h.at[pl.ds(crow, gk)], idx_v)
                lps = []
                for b in range(gk):
                    ebase = wid * ew + (g * gk + b) * chunk
                    sl = pl.ds(b * chunk, chunk)
                    lps.append((
                        pltpu.async_copy(pay_h.at[pl.ds(ebase, chunk)],
                                         buf.at[sl], ls[b]),
                        pltpu.async_copy(pay2_h.at[pl.ds(ebase, chunk)],
                                         buf2.at[sl], ls[b]),
                    ))
                sps = []
                for b in range(gk):
                    lps[b][0].wait()
                    lps[b][1].wait()
                    sl = pl.ds(b * chunk, chunk)
                    sps.append(pltpu.async_copy(
                        buf.at[sl], acc.at[idx_v.at[b]], ss[b], add=True))
                    sps.append(pltpu.async_copy(
                        buf2.at[sl], acc2.at[idx_v.at[b]], ss[b], add=True))
                for s_ in sps:
                    s_.wait()
                return carry

            lax.fori_loop(0, ngroup, body, 0)

        plsc.subcore_barrier()
        pltpu.sync_copy(acc.at[rsl], out_h.at[cid, rsl])
        pltpu.sync_copy(acc2.at[rsl], out2_h.at[cid, rsl])

    return scatter


def _embed_body(x_ref, cp_ref, wemb, bemb, wr, br, wc,
                h_ref, a_ref, b_ref, ca_ref, cb_ref):
    hv = jnp.dot(x_ref[...], wemb[...], preferred_element_type=_F32) + bemb[...]
    cpv = cp_ref[...]
    h_ref[...] = hv
    a_ref[...] = jnp.dot(hv, wr[...], preferred_element_type=_F32) + br[...]
    b_ref[...] = jnp.dot(hv, wc[...], preferred_element_type=_F32)
    ca_ref[...] = cpv
    cb_ref[...] = -cpv


def _tc_embed(x, cp, wemb, bemb, wr, br, wc):
    return pl.pallas_call(
        _embed_body,
        grid=(N // BN,),
        in_specs=[pl.BlockSpec((BN, H), lambda i: (i, 0)),
                  pl.BlockSpec((BN, CP), lambda i: (i, 0)),
                  _full_spec((H, H)), _full_spec((1, H)),
                  _full_spec((H, H)), _full_spec((1, H)),
                  _full_spec((H, H))],
        out_specs=[pl.BlockSpec((BN, H), lambda i: (i, 0)),
                   pl.BlockSpec((BN, H), lambda i: (i, 0)),
                   pl.BlockSpec((BN, H), lambda i: (i, 0)),
                   pl.BlockSpec((BN, CP), lambda i: (i, 0)),
                   pl.BlockSpec((BN, CP), lambda i: (i, 0))],
        out_shape=[jax.ShapeDtypeStruct((N, H), _F32),
                   jax.ShapeDtypeStruct((N, H), _F32),
                   jax.ShapeDtypeStruct((N, H), _F32),
                   jax.ShapeDtypeStruct((N, CP), _F32),
                   jax.ShapeDtypeStruct((N, CP), _F32)],
    )(x, cp, wemb, bemb, wr, br, wc)


def _edge_body(g1_ref, g2_ref, cd_ref, ea_ref, wrad, wea, w2, b2, watt, batt,
               wc1, bc1, wc2, out_ref, out2_ref):
    hsum = g1_ref[...] + g2_ref[...]
    cd = cd_ref[...]
    radial = jnp.sum(cd * cd, axis=1, keepdims=True)
    t = hsum + radial * wrad[...] + jnp.dot(
        ea_ref[...], wea[...], preferred_element_type=_F32)
    m = _silu(t)
    m = _silu(jnp.dot(m, w2[...], preferred_element_type=_F32) + b2[...])
    att = jax.nn.sigmoid(
        jnp.sum(m * watt[...], axis=1, keepdims=True) + batt[0, 0])
    m = m * att
    cmid = _silu(jnp.dot(m, wc1[...], preferred_element_type=_F32) + bc1[...])
    cval = jnp.sum(cmid * wc2[...], axis=1, keepdims=True)
    scale = cval / (jnp.sqrt(radial) + EPS)
    ones_col = (lax.broadcasted_iota(jnp.int32, (BE, CP), 1) == 3).astype(_F32)
    out_ref[...] = m
    out2_ref[...] = cd * scale + ones_col


def _tc_edge(g1, g2, cd, ea, wrad, wea, w2, b2, watt, batt, wc1, bc1, wc2):
    return pl.pallas_call(
        _edge_body,
        grid=(E // BE,),
        in_specs=[pl.BlockSpec((BE, H), lambda i: (i, 0)),
                  pl.BlockSpec((BE, H), lambda i: (i, 0)),
                  pl.BlockSpec((BE, CP), lambda i: (i, 0)),
                  pl.BlockSpec((BE, 4), lambda i: (i, 0)),
                  _full_spec((1, H)), _full_spec((4, H)),
                  _full_spec((H, H)), _full_spec((1, H)),
                  _full_spec((1, H)), _full_spec((1, 1)),
                  _full_spec((H, H)), _full_spec((1, H)),
                  _full_spec((1, H))],
        out_specs=[pl.BlockSpec((BE, H), lambda i: (i, 0)),
                   pl.BlockSpec((BE, CP), lambda i: (i, 0))],
        out_shape=[jax.ShapeDtypeStruct((E, H), _F32),
                   jax.ShapeDtypeStruct((E, CP), _F32)],
    )(g1, g2, cd, ea, wrad, wea, w2, b2, watt, batt, wc1, bc1, wc2)


def _node_common(acc_ref, acc2_ref, h_ref, w1a, w1b, b1, w2, b2):
    nagg = acc_ref[0] + acc_ref[1]
    ctail = acc2_ref[0] + acc2_ref[1]
    lane = lax.broadcasted_iota(jnp.int32, (BN, CP), 1)
    cnt = jnp.sum(jnp.where(lane == 3, ctail, 0.0), axis=1, keepdims=True)
    upd = jnp.where(lane < 3, ctail, 0.0) / jnp.maximum(cnt, 1.0)
    hv = h_ref[...]
    t = _silu(jnp.dot(hv, w1a[...], preferred_element_type=_F32)
              + jnp.dot(nagg, w1b[...], preferred_element_type=_F32)
              + b1[...])
    hnew = hv + jnp.dot(t, w2[...], preferred_element_type=_F32) + b2[...]
    return hnew, upd


def _node_body(acc_ref, acc2_ref, h_ref, cp_ref, w1a, w1b, b1, w2, b2,
               wrn, brn, wcn, ho_ref, co_ref, a_ref, bo_ref, ca_ref, cb_ref):
    hnew, upd = _node_common(acc_ref, acc2_ref, h_ref, w1a, w1b, b1, w2, b2)
    cnew = cp_ref[...] + upd
    ho_ref[...] = hnew
    co_ref[...] = cnew
    a_ref[...] = jnp.dot(hnew, wrn[...], preferred_element_type=_F32) + brn[...]
    bo_ref[...] = jnp.dot(hnew, wcn[...], preferred_element_type=_F32)
    ca_ref[...] = cnew
    cb_ref[...] = -cnew


def _tc_node(acc, acc2, h, cp, w1a, w1b, b1, w2, b2, wrn, brn, wcn):
    return pl.pallas_call(
        _node_body,
        grid=(N // BN,),
        in_specs=[pl.BlockSpec((NC, BN, H), lambda i: (0, i, 0)),
                  pl.BlockSpec((NC, BN, CP), lambda i: (0, i, 0)),
                  pl.BlockSpec((BN, H), lambda i: (i, 0)),
                  pl.BlockSpec((BN, CP), lambda i: (i, 0)),
                  _full_spec((H, H)), _full_spec((H, H)), _full_spec((1, H)),
                  _full_spec((H, H)), _full_spec((1, H)),
                  _full_spec((H, H)), _full_spec((1, H)), _full_spec((H, H))],
        out_specs=[pl.BlockSpec((BN, H), lambda i: (i, 0)),
                   pl.BlockSpec((BN, CP), lambda i: (i, 0)),
                   pl.BlockSpec((BN, H), lambda i: (i, 0)),
                   pl.BlockSpec((BN, H), lambda i: (i, 0)),
                   pl.BlockSpec((BN, CP), lambda i: (i, 0)),
                   pl.BlockSpec((BN, CP), lambda i: (i, 0))],
        out_shape=[jax.ShapeDtypeStruct((N, H), _F32),
                   jax.ShapeDtypeStruct((N, CP), _F32),
                   jax.ShapeDtypeStruct((N, H), _F32),
                   jax.ShapeDtypeStruct((N, H), _F32),
                   jax.ShapeDtypeStruct((N, CP), _F32),
                   jax.ShapeDtypeStruct((N, CP), _F32)],
    )(acc, acc2, h, cp, w1a, w1b, b1, w2, b2, wrn, brn, wcn)


def _final_body(acc_ref, acc2_ref, h_ref, w1a, w1b, b1, w2, b2, weo, beo,
                nf_ref):
    hnew, _ = _node_common(acc_ref, acc2_ref, h_ref, w1a, w1b, b1, w2, b2)
    nf_ref[...] = jnp.dot(hnew, weo[...], preferred_element_type=_F32) + beo[...]


def _tc_final(acc, acc2, h, w1a, w1b, b1, w2, b2, weo, beo):
    return pl.pallas_call(
        _final_body,
        grid=(N // BN,),
        in_specs=[pl.BlockSpec((NC, BN, H), lambda i: (0, i, 0)),
                  pl.BlockSpec((NC, BN, CP), lambda i: (0, i, 0)),
                  pl.BlockSpec((BN, H), lambda i: (i, 0)),
                  _full_spec((H, H)), _full_spec((H, H)), _full_spec((1, H)),
                  _full_spec((H, H)), _full_spec((1, H)),
                  _full_spec((H, H)), _full_spec((1, H))],
        out_specs=pl.BlockSpec((BN, H), lambda i: (i, 0)),
        out_shape=jax.ShapeDtypeStruct((N, H), _F32),
    )(acc, acc2, h, w1a, w1b, b1, w2, b2, weo, beo)


def _pool_body(acc_ref, acc2_ref, w1, b1, w2, b2, out_ref):
    gsum = acc_ref[0] + acc_ref[1]
    tail = acc2_ref[0] + acc2_ref[1]
    lane = lax.broadcasted_iota(jnp.int32, (G, CP), 1)
    cnt = jnp.sum(jnp.where(lane == 3, tail, 0.0), axis=1, keepdims=True)
    pooled = gsum / jnp.maximum(cnt, 1.0)
    o = _silu(jnp.dot(pooled, w1[...], preferred_element_type=_F32) + b1[...])
    out_ref[...] = jnp.sum(o * w2[...], axis=1, keepdims=True) + b2[0, 0]


def _tc_pool(acc, acc2, w1, b1, w2, b2):
    return pl.pallas_call(
        _pool_body,
        grid=(1,),
        in_specs=[_full_spec((NC, G, H)), _full_spec((NC, G, CP)),
                  _full_spec((H, H)), _full_spec((1, H)),
                  _full_spec((1, H)), _full_spec((1, 1))],
        out_specs=_full_spec((G, 1)),
        out_shape=jax.ShapeDtypeStruct((G, 1), _F32),
    )(acc, acc2, w1, b1, w2, b2)


def kernel(x, pos, edge_index, edge_attr, batch, params):
    row2 = edge_index[0].reshape(E // CHUNK, CHUNK)
    col2 = edge_index[1].reshape(E // CHUNK, CHUNK)
    rows2 = edge_index[0].reshape(E // SCHUNK, SCHUNK)
    batch2 = batch.reshape(N // SCHUNK, SCHUNK)
    cp0 = jnp.pad(pos, ((0, 0), (0, CP - 3)))
    zeros_n = jnp.zeros((N, H), _F32)
    zeros_n2 = jnp.zeros((N, CP), _F32)
    zeros_g = jnp.zeros((G, H), _F32)
    zeros_g2 = jnp.zeros((G, CP), _F32)
    ones_n2 = jnp.zeros((N, CP), _F32).at[:, 3].set(1.0)

    layers = params['layers']

    def e1_split(lp):
        w = lp['e1']['W']
        return (w[:H], w[H:2 * H], w[2 * H:2 * H + 1], w[2 * H + 1:],
                lp['e1']['b'].reshape(1, H))

    sc_gather = _make_sc_gather()
    edge_scatter = _make_sc_scatter(E, N, SCHUNK, NW)
    pool_scatter = _make_sc_scatter(N, G, SCHUNK, 25)

    wr0, wc0, _, _, br0 = e1_split(layers[0])
    h, ta, tb, cta, ctb = _tc_embed(
        x, cp0, params['emb_in']['W'], params['emb_in']['b'].reshape(1, H),
        wr0, br0, wc0)
    coord = cp0

    nf = None
    for l in range(N_LAYERS):
        lp = layers[l]
        _, _, wrad, wea, _ = e1_split(lp)
        g1, g2, cd = sc_gather(ta, tb, cta, ctb, row2, col2)
        pay, pay2 = _tc_edge(
            g1, g2, cd, edge_attr, wrad, wea,
            lp['e2']['W'], lp['e2']['b'].reshape(1, H),
            lp['att']['W'].reshape(1, H), lp['att']['b'].reshape(1, 1),
            lp['c1']['W'], lp['c1']['b'].reshape(1, H),
            lp['c2']['W'].reshape(1, H))
        acc, acc2 = edge_scatter(pay, pay2, rows2, zeros_n, zeros_n2)
        n1w = lp['n1']['W']
        nodew = (n1w[:H], n1w[H:], lp['n1']['b'].reshape(1, H),
                 lp['n2']['W'], lp['n2']['b'].reshape(1, H))
        if l < N_LAYERS - 1:
            wrn, wcn, _, _, brn = e1_split(layers[l + 1])
            h, coord, ta, tb, cta, ctb = _tc_node(
                acc, acc2, h, coord, *nodew, wrn, brn, wcn)
        else:
            nf = _tc_final(acc, acc2, h, *nodew,
                           params['emb_out']['W'],
                           params['emb_out']['b'].reshape(1, H))

    pacc, pacc2 = pool_scatter(nf, ones_n2, batch2, zeros_g, zeros_g2)
    return _tc_pool(
        pacc, pacc2, params['out1']['W'], params['out1']['b'].reshape(1, H),
        params['out2']['W'].reshape(1, H), params['out2']['b'].reshape(1, 1))


# speedup vs baseline: 4.3245x; 1.1083x over previous
import functools

import jax
import jax.numpy as jnp
from jax import lax
from jax.experimental import pallas as pl
from jax.experimental.pallas import tpu as pltpu
from jax.experimental.pallas import tpu_sc as plsc

N = 10000
E = 320000
H = 128
G = 64
CP = 16
EPS = 1e-8
N_LAYERS = 4

NC = 2
NS = 16
NW = NC * NS
CHUNK = 40
GK = 5
EH = E // 2

BN = 1000
BE = 4000

_F32 = jnp.float32


def _silu(v):
    return v * jax.nn.sigmoid(v)


def _full_spec(shape):
    return pl.BlockSpec(shape, lambda *_: tuple(0 for _ in shape))


def _sc_mesh():
    return plsc.VectorSubcoreMesh(
        core_axis_name="c", subcore_axis_name="s",
        num_cores=NC, num_subcores=NS)


def _make_sc_gather(e_items, chunk):
    ew = e_items // NW
    nchunk = ew // chunk

    @functools.partial(
        pl.kernel,
        out_type=(jax.ShapeDtypeStruct((e_items, H), _F32),
                  jax.ShapeDtypeStruct((e_items, H), _F32),
                  jax.ShapeDtypeStruct((e_items, CP), _F32)),
        mesh=_sc_mesh(),
        scratch_types=(pltpu.VMEM((GK, chunk), jnp.int32),
                       pltpu.VMEM((GK, chunk), jnp.int32),
                       pltpu.VMEM((GK * chunk, H), _F32),
                       pltpu.VMEM((GK * chunk, H), _F32),
                       pltpu.VMEM((GK * chunk, CP), _F32),
                       pltpu.VMEM((GK * chunk, CP), _F32),
                       pltpu.VMEM((GK * chunk, CP), _F32))
                      + (pltpu.SemaphoreType.DMA,) * (2 * GK),
        compiler_params=pltpu.CompilerParams(use_tc_tiling_on_sc=False),
    )
    def gather(ta, tb, cta, ctb, row2, col2, g1, g2, cd,
               idxr, idxc, bufa, bufb, bufca, bufcb, bufcd, *sems):
        CHUNK = chunk
        gs, ws = sems[:GK], sems[GK:]
        wid = lax.axis_index("s") * NC + lax.axis_index("c")
        base0 = wid * ew
        crow0 = wid * nchunk

        def body(g, carry):
            crow = crow0 + g * GK
            pltpu.sync_copy(row2.at[pl.ds(crow, GK)], idxr)
            pltpu.sync_copy(col2.at[pl.ds(crow, GK)], idxc)
            cps = []
            for b in range(GK):
                sl = pl.ds(b * CHUNK, CHUNK)
                cps.append((
                    pltpu.async_copy(ta.at[idxr.at[b]], bufa.at[sl], gs[b]),
                    pltpu.async_copy(tb.at[idxc.at[b]], bufb.at[sl], gs[b]),
                    pltpu.async_copy(cta.at[idxr.at[b]], bufca.at[sl], gs[b]),
                    pltpu.async_copy(ctb.at[idxc.at[b]], bufcb.at[sl], gs[b]),
                ))
            wps = []
            for b in range(GK):
                for c in cps[b]:
                    c.wait()
                for i in range(b * CHUNK, (b + 1) * CHUNK):
                    bufcd[i, :] = bufca[i, :] + bufcb[i, :]
                sl = pl.ds(b * CHUNK, CHUNK)
                ebase = base0 + (g * GK + b) * CHUNK
                esl = pl.ds(ebase, CHUNK)
                wps.append(pltpu.async_copy(bufa.at[sl], g1.at[esl], ws[b]))
                wps.append(pltpu.async_copy(bufb.at[sl], g2.at[esl], ws[b]))
                wps.append(pltpu.async_copy(bufcd.at[sl], cd.at[esl], ws[b]))
            for w in wps:
                w.wait()
            return carry

        lax.fori_loop(0, nchunk // GK, body, 0)

    return gather


def _make_sc_scatter(n_items, n_rows, chunk, workers):
    ew = n_items // workers
    nchunk = ew // chunk
    gk = min(GK, nchunk)
    ngroup = nchunk // gk
    rps = n_rows // NS

    @functools.partial(
        pl.kernel,
        out_type=(jax.ShapeDtypeStruct((NC, n_rows, H), _F32),
                  jax.ShapeDtypeStruct((NC, n_rows, CP), _F32)),
        mesh=_sc_mesh(),
        scratch_types=(pltpu.VMEM((gk, chunk), jnp.int32),
                       pltpu.VMEM((gk * chunk, H), _F32),
                       pltpu.VMEM((gk * chunk, CP), _F32))
                      + (pltpu.SemaphoreType.DMA,) * (2 * gk)
                      + (pltpu.VMEM_SHARED((n_rows, H), _F32),
                         pltpu.VMEM_SHARED((n_rows, CP), _F32)),
        compiler_params=pltpu.CompilerParams(use_tc_tiling_on_sc=False),
    )
    def scatter(pay_h, pay2_h, idx2_h, zero_h, zero2_h, out_h, out2_h,
                idx_v, buf, buf2, *rest):
        ls, ss = rest[:gk], rest[gk:2 * gk]
        acc, acc2 = rest[2 * gk], rest[2 * gk + 1]
        cid = lax.axis_index("c")
        sid = lax.axis_index("s")
        wid = sid * NC + cid
        rsl = pl.ds(sid * rps, rps)
        pltpu.sync_copy(zero_h.at[rsl], acc.at[rsl])
        pltpu.sync_copy(zero2_h.at[rsl], acc2.at[rsl])
        plsc.subcore_barrier()

        @pl.when(wid < workers)
        def _():
            def body(g, carry):
                crow = wid * nchunk + g * gk
                pltpu.sync_copy(idx2_h.at[pl.ds(crow, gk)], idx_v)
                lps = []
                for b in range(gk):
                    ebase = wid * ew + (g * gk + b) * chunk
                    sl = pl.ds(b * chunk, chunk)
                    lps.append((
                        pltpu.async_copy(pay_h.at[pl.ds(ebase, chunk)],
                                         buf.at[sl], ls[b]),
                        pltpu.async_copy(pay2_h.at[pl.ds(ebase, chunk)],
                                         buf2.at[sl], ls[b]),
                    ))
                sps = []
                for b in range(gk):
                    lps[b][0].wait()
                    lps[b][1].wait()
                    sl = pl.ds(b * chunk, chunk)
                    sps.append(pltpu.async_copy(
                        buf.at[sl], acc.at[idx_v.at[b]], ss[b], add=True))
                    sps.append(pltpu.async_copy(
                        buf2.at[sl], acc2.at[idx_v.at[b]], ss[b], add=True))
                for s_ in sps:
                    s_.wait()
                return carry

            lax.fori_loop(0, ngroup, body, 0)

        plsc.subcore_barrier()
        pltpu.sync_copy(acc.at[rsl], out_h.at[cid, rsl])
        pltpu.sync_copy(acc2.at[rsl], out2_h.at[cid, rsl])

    return scatter


def _embed_body(x_ref, cp_ref, wemb, bemb, wr, br, wc,
                h_ref, a_ref, b_ref, ca_ref, cb_ref):
    hv = jnp.dot(x_ref[...], wemb[...], preferred_element_type=_F32) + bemb[...]
    cpv = cp_ref[...]
    h_ref[...] = hv
    a_ref[...] = jnp.dot(hv, wr[...], preferred_element_type=_F32) + br[...]
    b_ref[...] = jnp.dot(hv, wc[...], preferred_element_type=_F32)
    ca_ref[...] = cpv
    cb_ref[...] = -cpv


def _tc_embed(x, cp, wemb, bemb, wr, br, wc):
    return pl.pallas_call(
        _embed_body,
        grid=(N // BN,),
        in_specs=[pl.BlockSpec((BN, H), lambda i: (i, 0)),
                  pl.BlockSpec((BN, CP), lambda i: (i, 0)),
                  _full_spec((H, H)), _full_spec((1, H)),
                  _full_spec((H, H)), _full_spec((1, H)),
                  _full_spec((H, H))],
        out_specs=[pl.BlockSpec((BN, H), lambda i: (i, 0)),
                   pl.BlockSpec((BN, H), lambda i: (i, 0)),
                   pl.BlockSpec((BN, H), lambda i: (i, 0)),
                   pl.BlockSpec((BN, CP), lambda i: (i, 0)),
                   pl.BlockSpec((BN, CP), lambda i: (i, 0))],
        out_shape=[jax.ShapeDtypeStruct((N, H), _F32),
                   jax.ShapeDtypeStruct((N, H), _F32),
                   jax.ShapeDtypeStruct((N, H), _F32),
                   jax.ShapeDtypeStruct((N, CP), _F32),
                   jax.ShapeDtypeStruct((N, CP), _F32)],
    )(x, cp, wemb, bemb, wr, br, wc)


def _edge_body(g1_ref, g2_ref, cd_ref, ea_ref, wrad, wea, w2, b2, watt, batt,
               wc1, bc1, wc2, out_ref, out2_ref):
    hsum = g1_ref[...] + g2_ref[...]
    cd = cd_ref[...]
    radial = jnp.sum(cd * cd, axis=1, keepdims=True)
    t = hsum + radial * wrad[...] + jnp.dot(
        ea_ref[...], wea[...], preferred_element_type=_F32)
    m = _silu(t)
    m = _silu(jnp.dot(m, w2[...], preferred_element_type=_F32) + b2[...])
    att = jax.nn.sigmoid(
        jnp.sum(m * watt[...], axis=1, keepdims=True) + batt[0, 0])
    m = m * att
    cmid = _silu(jnp.dot(m, wc1[...], preferred_element_type=_F32) + bc1[...])
    cval = jnp.sum(cmid * wc2[...], axis=1, keepdims=True)
    scale = cval / (jnp.sqrt(radial) + EPS)
    ones_col = (lax.broadcasted_iota(jnp.int32, (BE, CP), 1) == 3).astype(_F32)
    out_ref[...] = m
    out2_ref[...] = cd * scale + ones_col


def _tc_edge(g1, g2, cd, ea, wrad, wea, w2, b2, watt, batt, wc1, bc1, wc2):
    e_items = g1.shape[0]
    return pl.pallas_call(
        _edge_body,
        grid=(e_items // BE,),
        in_specs=[pl.BlockSpec((BE, H), lambda i: (i, 0)),
                  pl.BlockSpec((BE, H), lambda i: (i, 0)),
                  pl.BlockSpec((BE, CP), lambda i: (i, 0)),
                  pl.BlockSpec((BE, 4), lambda i: (i, 0)),
                  _full_spec((1, H)), _full_spec((4, H)),
                  _full_spec((H, H)), _full_spec((1, H)),
                  _full_spec((1, H)), _full_spec((1, 1)),
                  _full_spec((H, H)), _full_spec((1, H)),
                  _full_spec((1, H))],
        out_specs=[pl.BlockSpec((BE, H), lambda i: (i, 0)),
                   pl.BlockSpec((BE, CP), lambda i: (i, 0))],
        out_shape=[jax.ShapeDtypeStruct((e_items, H), _F32),
                   jax.ShapeDtypeStruct((e_items, CP), _F32)],
    )(g1, g2, cd, ea, wrad, wea, w2, b2, watt, batt, wc1, bc1, wc2)


def _node_common(acc_ref, acc2_ref, accb_ref, acc2b_ref, h_ref,
                 w1a, w1b, b1, w2, b2):
    nagg = acc_ref[0] + acc_ref[1] + accb_ref[0] + accb_ref[1]
    ctail = acc2_ref[0] + acc2_ref[1] + acc2b_ref[0] + acc2b_ref[1]
    lane = lax.broadcasted_iota(jnp.int32, (BN, CP), 1)
    cnt = jnp.sum(jnp.where(lane == 3, ctail, 0.0), axis=1, keepdims=True)
    upd = jnp.where(lane < 3, ctail, 0.0) / jnp.maximum(cnt, 1.0)
    hv = h_ref[...]
    t = _silu(jnp.dot(hv, w1a[...], preferred_element_type=_F32)
              + jnp.dot(nagg, w1b[...], preferred_element_type=_F32)
              + b1[...])
    hnew = hv + jnp.dot(t, w2[...], preferred_element_type=_F32) + b2[...]
    return hnew, upd


def _node_body(acc_ref, acc2_ref, accb_ref, acc2b_ref, h_ref, cp_ref,
               w1a, w1b, b1, w2, b2,
               wrn, brn, wcn, ho_ref, co_ref, a_ref, bo_ref, ca_ref, cb_ref):
    hnew, upd = _node_common(acc_ref, acc2_ref, accb_ref, acc2b_ref, h_ref,
                             w1a, w1b, b1, w2, b2)
    cnew = cp_ref[...] + upd
    ho_ref[...] = hnew
    co_ref[...] = cnew
    a_ref[...] = jnp.dot(hnew, wrn[...], preferred_element_type=_F32) + brn[...]
    bo_ref[...] = jnp.dot(hnew, wcn[...], preferred_element_type=_F32)
    ca_ref[...] = cnew
    cb_ref[...] = -cnew


def _tc_node(acc, acc2, accb, acc2b, h, cp, w1a, w1b, b1, w2, b2,
             wrn, brn, wcn):
    return pl.pallas_call(
        _node_body,
        grid=(N // BN,),
        in_specs=[pl.BlockSpec((NC, BN, H), lambda i: (0, i, 0)),
                  pl.BlockSpec((NC, BN, CP), lambda i: (0, i, 0)),
                  pl.BlockSpec((NC, BN, H), lambda i: (0, i, 0)),
                  pl.BlockSpec((NC, BN, CP), lambda i: (0, i, 0)),
                  pl.BlockSpec((BN, H), lambda i: (i, 0)),
                  pl.BlockSpec((BN, CP), lambda i: (i, 0)),
                  _full_spec((H, H)), _full_spec((H, H)), _full_spec((1, H)),
                  _full_spec((H, H)), _full_spec((1, H)),
                  _full_spec((H, H)), _full_spec((1, H)), _full_spec((H, H))],
        out_specs=[pl.BlockSpec((BN, H), lambda i: (i, 0)),
                   pl.BlockSpec((BN, CP), lambda i: (i, 0)),
                   pl.BlockSpec((BN, H), lambda i: (i, 0)),
                   pl.BlockSpec((BN, H), lambda i: (i, 0)),
                   pl.BlockSpec((BN, CP), lambda i: (i, 0)),
                   pl.BlockSpec((BN, CP), lambda i: (i, 0))],
        out_shape=[jax.ShapeDtypeStruct((N, H), _F32),
                   jax.ShapeDtypeStruct((N, CP), _F32),
                   jax.ShapeDtypeStruct((N, H), _F32),
                   jax.ShapeDtypeStruct((N, H), _F32),
                   jax.ShapeDtypeStruct((N, CP), _F32),
                   jax.ShapeDtypeStruct((N, CP), _F32)],
    )(acc, acc2, accb, acc2b, h, cp, w1a, w1b, b1, w2, b2, wrn, brn, wcn)


def _final_body(acc_ref, acc2_ref, accb_ref, acc2b_ref, h_ref,
                w1a, w1b, b1, w2, b2, weo, beo, nf_ref):
    hnew, _ = _node_common(acc_ref, acc2_ref, accb_ref, acc2b_ref, h_ref,
                           w1a, w1b, b1, w2, b2)
    nf_ref[...] = jnp.dot(hnew, weo[...], preferred_element_type=_F32) + beo[...]


def _tc_final(acc, acc2, accb, acc2b, h, w1a, w1b, b1, w2, b2, weo, beo):
    return pl.pallas_call(
        _final_body,
        grid=(N // BN,),
        in_specs=[pl.BlockSpec((NC, BN, H), lambda i: (0, i, 0)),
                  pl.BlockSpec((NC, BN, CP), lambda i: (0, i, 0)),
                  pl.BlockSpec((NC, BN, H), lambda i: (0, i, 0)),
                  pl.BlockSpec((NC, BN, CP), lambda i: (0, i, 0)),
                  pl.BlockSpec((BN, H), lambda i: (i, 0)),
                  _full_spec((H, H)), _full_spec((H, H)), _full_spec((1, H)),
                  _full_spec((H, H)), _full_spec((1, H)),
                  _full_spec((H, H)), _full_spec((1, H))],
        out_specs=pl.BlockSpec((BN, H), lambda i: (i, 0)),
        out_shape=jax.ShapeDtypeStruct((N, H), _F32),
    )(acc, acc2, accb, acc2b, h, w1a, w1b, b1, w2, b2, weo, beo)


def _pool_body(acc_ref, acc2_ref, w1, b1, w2, b2, out_ref):
    gsum = acc_ref[0] + acc_ref[1]
    tail = acc2_ref[0] + acc2_ref[1]
    lane = lax.broadcasted_iota(jnp.int32, (G, CP), 1)
    cnt = jnp.sum(jnp.where(lane == 3, tail, 0.0), axis=1, keepdims=True)
    pooled = gsum / jnp.maximum(cnt, 1.0)
    o = _silu(jnp.dot(pooled, w1[...], preferred_element_type=_F32) + b1[...])
    out_ref[...] = jnp.sum(o * w2[...], axis=1, keepdims=True) + b2[0, 0]


def _tc_pool(acc, acc2, w1, b1, w2, b2):
    return pl.pallas_call(
        _pool_body,
        grid=(1,),
        in_specs=[_full_spec((NC, G, H)), _full_spec((NC, G, CP)),
                  _full_spec((H, H)), _full_spec((1, H)),
                  _full_spec((1, H)), _full_spec((1, 1))],
        out_specs=_full_spec((G, 1)),
        out_shape=jax.ShapeDtypeStruct((G, 1), _F32),
    )(acc, acc2, w1, b1, w2, b2)


def kernel(x, pos, edge_index, edge_attr, batch, params):
    row2 = edge_index[0].reshape(E // CHUNK, CHUNK)
    col2 = edge_index[1].reshape(E // CHUNK, CHUNK)
    hr = EH // CHUNK
    rowA, rowB = row2[:hr], row2[hr:]
    colA, colB = col2[:hr], col2[hr:]
    eaA, eaB = edge_attr[:EH], edge_attr[EH:]
    batch2 = batch.reshape(N // CHUNK, CHUNK)
    cp0 = jnp.pad(pos, ((0, 0), (0, CP - 3)))
    zeros_n = jnp.zeros((N, H), _F32)
    zeros_n2 = jnp.zeros((N, CP), _F32)
    zeros_g = jnp.zeros((G, H), _F32)
    zeros_g2 = jnp.zeros((G, CP), _F32)
    ones_n2 = jnp.zeros((N, CP), _F32).at[:, 3].set(1.0)

    layers = params['layers']

    def e1_split(lp):
        w = lp['e1']['W']
        return (w[:H], w[H:2 * H], w[2 * H:2 * H + 1], w[2 * H + 1:],
                lp['e1']['b'].reshape(1, H))

    sc_gather = _make_sc_gather(EH, CHUNK)
    edge_scatter = _make_sc_scatter(EH, N, CHUNK, NW)
    pool_scatter = _make_sc_scatter(N, G, CHUNK, 25)

    wr0, wc0, _, _, br0 = e1_split(layers[0])
    h, ta, tb, cta, ctb = _tc_embed(
        x, cp0, params['emb_in']['W'], params['emb_in']['b'].reshape(1, H),
        wr0, br0, wc0)
    coord = cp0

    nf = None
    for l in range(N_LAYERS):
        lp = layers[l]
        _, _, wrad, wea, _ = e1_split(lp)
        edgew = (wrad, wea,
                 lp['e2']['W'], lp['e2']['b'].reshape(1, H),
                 lp['att']['W'].reshape(1, H), lp['att']['b'].reshape(1, 1),
                 lp['c1']['W'], lp['c1']['b'].reshape(1, H),
                 lp['c2']['W'].reshape(1, H))
        g1A, g2A, cdA = sc_gather(ta, tb, cta, ctb, rowA, colA)
        g1B, g2B, cdB = sc_gather(ta, tb, cta, ctb, rowB, colB)
        payA, pay2A = _tc_edge(g1A, g2A, cdA, eaA, *edgew)
        payB, pay2B = _tc_edge(g1B, g2B, cdB, eaB, *edgew)
        accA, acc2A = edge_scatter(payA, pay2A, rowA, zeros_n, zeros_n2)
        accB, acc2B = edge_scatter(payB, pay2B, rowB, zeros_n, zeros_n2)
        n1w = lp['n1']['W']
        nodew = (n1w[:H], n1w[H:], lp['n1']['b'].reshape(1, H),
                 lp['n2']['W'], lp['n2']['b'].reshape(1, H))
        if l < N_LAYERS - 1:
            wrn, wcn, _, _, brn = e1_split(layers[l + 1])
            h, coord, ta, tb, cta, ctb = _tc_node(
                accA, acc2A, accB, acc2B, h, coord, *nodew, wrn, brn, wcn)
        else:
            nf = _tc_final(accA, acc2A, accB, acc2B, h, *nodew,
                           params['emb_out']['W'],
                           params['emb_out']['b'].reshape(1, H))

    pacc, pacc2 = pool_scatter(nf, ones_n2, batch2, zeros_g, zeros_g2)
    return _tc_pool(
        pacc, pacc2, params['out1']['W'], params['out1']['b'].reshape(1, H),
        params['out2']['W'].reshape(1, H), params['out2']['b'].reshape(1, 1))
